# Initial kernel scaffold; baseline (speedup 1.0000x reference)
import jax
import jax.numpy as jnp
from jax.experimental import pallas as pl


def kernel(x, gate_w, gate_weights, up_weights, down_weights):
    raise NotImplementedError("write your pallas kernel here")


# speedup vs baseline: 1.4749x; 1.4749x over previous
import functools

import jax
import jax.numpy as jnp
from jax.experimental import pallas as pl
from jax.experimental.pallas import tpu as pltpu

BM = 128
NFF = 2


def _ffn_body(blk_ref, exp_ref, lo_ref, hi_ref, first_ref,
              x_ref, wg_ref, wu_ref, wd_ref, o_ref):
    i = pl.program_id(0)
    j = pl.program_id(1)
    x = x_ref[...]
    g = jnp.dot(x, wg_ref[0], preferred_element_type=jnp.float32)
    u = jnp.dot(x, wu_ref[0], preferred_element_type=jnp.float32)
    h = (g * jax.nn.sigmoid(g)) * u
    y = jnp.dot(h, wd_ref[0], preferred_element_type=jnp.float32)
    rows = jax.lax.broadcasted_iota(jnp.int32, y.shape, 0)
    y = jnp.where((rows >= lo_ref[i]) & (rows < hi_ref[i]), y, 0.0)

    @pl.when((first_ref[i] == 1) & (j == 0))
    def _init():
        o_ref[...] = y

    @pl.when((first_ref[i] == 0) | (j != 0))
    def _acc():
        o_ref[...] += y


def _grouped_ffn(xs, gate_weights, up_weights, down_weights,
                 wi_blk, wi_exp, wi_lo, wi_hi, wi_first, nw):
    p, embed = xs.shape
    e, _, ff = gate_weights.shape
    ffb = ff // NFF
    grid = (nw, NFF)
    out_shape = jax.ShapeDtypeStruct((p, embed), jnp.float32)
    grid_spec = pltpu.PrefetchScalarGridSpec(
        num_scalar_prefetch=5,
        grid=grid,
        in_specs=[
            pl.BlockSpec((BM, embed),
                         lambda i, j, blk, exp, lo, hi, first: (blk[i], 0)),
            pl.BlockSpec((1, embed, ffb),
                         lambda i, j, blk, exp, lo, hi, first: (exp[i], 0, j)),
            pl.BlockSpec((1, embed, ffb),
                         lambda i, j, blk, exp, lo, hi, first: (exp[i], 0, j)),
            pl.BlockSpec((1, ffb, embed),
                         lambda i, j, blk, exp, lo, hi, first: (exp[i], j, 0)),
        ],
        out_specs=pl.BlockSpec((BM, embed),
                               lambda i, j, blk, exp, lo, hi, first: (blk[i], 0)),
    )
    return pl.pallas_call(
        _ffn_body,
        grid_spec=grid_spec,
        out_shape=out_shape,
    )(wi_blk, wi_exp, wi_lo, wi_hi, wi_first,
      xs, gate_weights, up_weights, down_weights)


def kernel(x, gate_w, gate_weights, up_weights, down_weights):
    b, s, embed = x.shape
    e = gate_w.shape[1]
    t = b * s
    xf = x.reshape(t, embed)

    logits = xf @ gate_w
    probs = jax.nn.softmax(logits, axis=-1)
    top_k_probs, top_k_indices = jax.lax.top_k(probs, 2)
    topk = top_k_indices.shape[-1]
    p = t * topk

    nb = p // BM
    nw = nb + e - 1

    flat_e = top_k_indices.reshape(-1).astype(jnp.int32)
    sort_idx = jnp.argsort(flat_e).astype(jnp.int32)
    st = sort_idx // topk
    counts = jnp.bincount(flat_e, length=e).astype(jnp.int32)
    off = jnp.concatenate([jnp.zeros((1,), jnp.int32),
                           jnp.cumsum(counts).astype(jnp.int32)])
    fb = off[:-1] // BM
    lb = jnp.maximum(off[1:] - 1, 0) // BM
    nblk = jnp.where(counts > 0, lb - fb + 1, 0).astype(jnp.int32)
    istart = jnp.concatenate([jnp.zeros((1,), jnp.int32),
                              jnp.cumsum(nblk).astype(jnp.int32)])
    total = istart[e]
    wi_exp = jnp.repeat(jnp.arange(e, dtype=jnp.int32), nblk,
                        total_repeat_length=nw)
    valid = jnp.arange(nw, dtype=jnp.int32) < total
    wi_exp = jnp.where(valid, wi_exp, e - 1)
    wi_blk = fb[wi_exp] + jnp.arange(nw, dtype=jnp.int32) - istart[wi_exp]
    wi_blk = jnp.where(valid, wi_blk, nb - 1)
    row0 = wi_blk * BM
    lo_g = jnp.maximum(off[wi_exp], row0)
    hi_g = jnp.minimum(off[wi_exp + 1], row0 + BM)
    wi_lo = jnp.where(valid, lo_g - row0, 0).astype(jnp.int32)
    wi_hi = jnp.where(valid, hi_g - row0, 0).astype(jnp.int32)
    wi_first = jnp.where(valid & (wi_lo == 0), 1, 0).astype(jnp.int32)

    xs = xf[st]

    ys = _grouped_ffn(xs, gate_weights, up_weights, down_weights,
                      wi_blk, wi_exp, wi_lo, wi_hi, wi_first, nw)

    inv = jnp.zeros((p,), jnp.int32).at[sort_idx].set(
        jnp.arange(p, dtype=jnp.int32))
    yt = ys[inv].reshape(t, topk, embed)
    out = jnp.sum(yt * top_k_probs[..., None], axis=1)
    return out.reshape(b, s, embed)


# speedup vs baseline: 2.3976x; 1.6255x over previous
import functools

import jax
import jax.numpy as jnp
from jax.experimental import pallas as pl
from jax.experimental.pallas import tpu as pltpu

BM = 128
NFF = 2


def _ffn_body(blk_ref, exp_ref, lo_ref, hi_ref, first_ref,
              x_ref, wg_ref, wu_ref, wd_ref, o_ref):
    i = pl.program_id(0)
    x = x_ref[...]
    g = jnp.dot(x, wg_ref[0], preferred_element_type=jnp.float32)
    u = jnp.dot(x, wu_ref[0], preferred_element_type=jnp.float32)
    h = (g * jax.nn.sigmoid(g)) * u
    y = jnp.dot(h, wd_ref[0], preferred_element_type=jnp.float32)
    rows = jax.lax.broadcasted_iota(jnp.int32, y.shape, 0)
    y = jnp.where((rows >= lo_ref[i]) & (rows < hi_ref[i]), y, 0.0)

    @pl.when(first_ref[i] == 1)
    def _init():
        o_ref[...] = y

    @pl.when(first_ref[i] == 0)
    def _acc():
        o_ref[...] += y


def _grouped_ffn(xs, gate_weights, up_weights, down_weights,
                 wi_blk, wi_exp, wi_lo, wi_hi, wi_first, nw):
    p, embed = xs.shape
    e, _, ff = gate_weights.shape
    out_shape = jax.ShapeDtypeStruct((p, embed), jnp.float32)
    grid_spec = pltpu.PrefetchScalarGridSpec(
        num_scalar_prefetch=5,
        grid=(nw,),
        in_specs=[
            pl.BlockSpec((BM, embed),
                         lambda i, blk, exp, lo, hi, first: (blk[i], 0)),
            pl.BlockSpec((1, embed, ff),
                         lambda i, blk, exp, lo, hi, first: (exp[i], 0, 0)),
            pl.BlockSpec((1, embed, ff),
                         lambda i, blk, exp, lo, hi, first: (exp[i], 0, 0)),
            pl.BlockSpec((1, ff, embed),
                         lambda i, blk, exp, lo, hi, first: (exp[i], 0, 0)),
        ],
        out_specs=pl.BlockSpec((BM, embed),
                               lambda i, blk, exp, lo, hi, first: (blk[i], 0)),
    )
    return pl.pallas_call(
        _ffn_body,
        grid_spec=grid_spec,
        out_shape=out_shape,
    )(wi_blk, wi_exp, wi_lo, wi_hi, wi_first,
      xs, gate_weights, up_weights, down_weights)


def kernel(x, gate_w, gate_weights, up_weights, down_weights):
    b, s, embed = x.shape
    e = gate_w.shape[1]
    t = b * s
    xf = x.reshape(t, embed)

    logits = xf @ gate_w
    probs = jax.nn.softmax(logits, axis=-1)
    top_k_probs, top_k_indices = jax.lax.top_k(probs, 2)
    topk = top_k_indices.shape[-1]
    p = t * topk

    nb = p // BM
    nw = nb + e - 1

    flat_e = top_k_indices.reshape(-1).astype(jnp.int32)
    sort_idx = jnp.argsort(flat_e).astype(jnp.int32)
    st = sort_idx // topk
    counts = jnp.bincount(flat_e, length=e).astype(jnp.int32)
    off = jnp.concatenate([jnp.zeros((1,), jnp.int32),
                           jnp.cumsum(counts).astype(jnp.int32)])
    fb = off[:-1] // BM
    lb = jnp.maximum(off[1:] - 1, 0) // BM
    nblk = jnp.where(counts > 0, lb - fb + 1, 0).astype(jnp.int32)
    istart = jnp.concatenate([jnp.zeros((1,), jnp.int32),
                              jnp.cumsum(nblk).astype(jnp.int32)])
    total = istart[e]
    wi_exp = jnp.repeat(jnp.arange(e, dtype=jnp.int32), nblk,
                        total_repeat_length=nw)
    valid = jnp.arange(nw, dtype=jnp.int32) < total
    wi_exp = jnp.where(valid, wi_exp, e - 1)
    wi_blk = fb[wi_exp] + jnp.arange(nw, dtype=jnp.int32) - istart[wi_exp]
    wi_blk = jnp.where(valid, wi_blk, nb - 1)
    row0 = wi_blk * BM
    lo_g = jnp.maximum(off[wi_exp], row0)
    hi_g = jnp.minimum(off[wi_exp + 1], row0 + BM)
    wi_lo = jnp.where(valid, lo_g - row0, 0).astype(jnp.int32)
    wi_hi = jnp.where(valid, hi_g - row0, 0).astype(jnp.int32)
    wi_first = jnp.where(valid & (wi_lo == 0), 1, 0).astype(jnp.int32)

    xs = xf[st]

    ys = _grouped_ffn(xs, gate_weights, up_weights, down_weights,
                      wi_blk, wi_exp, wi_lo, wi_hi, wi_first, nw)

    inv = jnp.zeros((p,), jnp.int32).at[sort_idx].set(
        jnp.arange(p, dtype=jnp.int32))
    yt = ys[inv].reshape(t, topk, embed)
    out = jnp.sum(yt * top_k_probs[..., None], axis=1)
    return out.reshape(b, s, embed)


# speedup vs baseline: 2.9341x; 1.2238x over previous
import jax
import jax.numpy as jnp
from jax import lax
from jax.experimental import pallas as pl
from jax.experimental.pallas import tpu as pltpu
from jax.experimental.pallas import tpu_sc as plsc

BM = 128
_NC, _NS = 2, 16
_NSC = _NC * _NS


def _ffn_body(blk_ref, exp_ref, lo_ref, hi_ref, first_ref,
              x_ref, sp_ref, wg_ref, wu_ref, wd_ref, o_ref):
    i = pl.program_id(0)
    x = x_ref[...]
    g = jnp.dot(x, wg_ref[0], preferred_element_type=jnp.float32)
    u = jnp.dot(x, wu_ref[0], preferred_element_type=jnp.float32)
    h = (g * jax.nn.sigmoid(g)) * u
    y = jnp.dot(h, wd_ref[0], preferred_element_type=jnp.float32)
    y = y * sp_ref[...]
    rows = jax.lax.broadcasted_iota(jnp.int32, y.shape, 0)
    y = jnp.where((rows >= lo_ref[i]) & (rows < hi_ref[i]), y, 0.0)

    @pl.when(first_ref[i] == 1)
    def _init():
        o_ref[...] = y

    @pl.when(first_ref[i] == 0)
    def _acc():
        o_ref[...] += y


def _grouped_ffn(xs, sp, gate_weights, up_weights, down_weights,
                 wi_blk, wi_exp, wi_lo, wi_hi, wi_first, nw):
    p, embed = xs.shape
    e, _, ff = gate_weights.shape
    out_shape = jax.ShapeDtypeStruct((p, embed), jnp.float32)
    grid_spec = pltpu.PrefetchScalarGridSpec(
        num_scalar_prefetch=5,
        grid=(nw,),
        in_specs=[
            pl.BlockSpec((BM, embed),
                         lambda i, blk, exp, lo, hi, first: (blk[i], 0)),
            pl.BlockSpec((BM, 1),
                         lambda i, blk, exp, lo, hi, first: (blk[i], 0)),
            pl.BlockSpec((1, embed, ff),
                         lambda i, blk, exp, lo, hi, first: (exp[i], 0, 0)),
            pl.BlockSpec((1, embed, ff),
                         lambda i, blk, exp, lo, hi, first: (exp[i], 0, 0)),
            pl.BlockSpec((1, ff, embed),
                         lambda i, blk, exp, lo, hi, first: (exp[i], 0, 0)),
        ],
        out_specs=pl.BlockSpec((BM, embed),
                               lambda i, blk, exp, lo, hi, first: (blk[i], 0)),
    )
    return pl.pallas_call(
        _ffn_body,
        grid_spec=grid_spec,
        out_shape=out_shape,
    )(wi_blk, wi_exp, wi_lo, wi_hi, wi_first,
      xs, sp, gate_weights, up_weights, down_weights)


_GCH = 32


def _sc_gather(table, idx):
    n = idx.shape[0]
    d = table.shape[1]
    rw = n // _NSC
    nch = rw // _GCH
    idx2 = idx.reshape(n // _GCH, _GCH)
    mesh = plsc.VectorSubcoreMesh(core_axis_name="c", subcore_axis_name="s",
                                  num_cores=_NC, num_subcores=_NS)

    def body(tab_hbm, idx_hbm, out_hbm, idx_v, buf0, buf1,
             g0, g1, o0, o1):
        wid = lax.axis_index("s") * _NC + lax.axis_index("c")
        base = wid * rw
        pltpu.sync_copy(idx_hbm.at[pl.ds(wid * nch, nch)], idx_v)
        bufs = (buf0, buf1)
        gsems = (g0, g1)
        osems = (o0, o1)
        gd = [None, None]
        od = [None, None]
        gd[0] = pltpu.async_copy(tab_hbm.at[idx_v.at[0]], bufs[0], gsems[0])
        for c in range(nch):
            b = c & 1
            nb = 1 - b
            gd[b].wait()
            if c >= 1:
                od[nb].wait()
            if c + 1 < nch:
                gd[nb] = pltpu.async_copy(tab_hbm.at[idx_v.at[c + 1]],
                                          bufs[nb], gsems[nb])
            od[b] = pltpu.async_copy(
                bufs[b], out_hbm.at[pl.ds(base + c * _GCH, _GCH)], osems[b])
        od[(nch - 1) & 1].wait()

    return pl.kernel(
        body,
        out_type=jax.ShapeDtypeStruct((n, d), jnp.float32),
        mesh=mesh,
        scratch_types=[
            pltpu.VMEM((nch, _GCH), jnp.int32),
            pltpu.VMEM((_GCH, d), jnp.float32),
            pltpu.VMEM((_GCH, d), jnp.float32),
            pltpu.SemaphoreType.DMA,
            pltpu.SemaphoreType.DMA,
            pltpu.SemaphoreType.DMA,
            pltpu.SemaphoreType.DMA,
        ],
    )(table, idx2)


_CCH = 32


def _sc_combine(ys, ia, ib):
    t = ia.shape[0]
    d = ys.shape[1]
    tw = t // _NSC
    ncc = tw // _CCH
    ia2 = ia.reshape(t // _CCH, _CCH)
    ib2 = ib.reshape(t // _CCH, _CCH)
    nvec = (_CCH * d) // 16
    cols = d // 16
    mesh = plsc.VectorSubcoreMesh(core_axis_name="c", subcore_axis_name="s",
                                  num_cores=_NC, num_subcores=_NS)

    def body(ys_hbm, ia_hbm, ib_hbm, out_hbm, ia_v, ib_v, ra, rb,
             ga, gb, osem):
        wid = lax.axis_index("s") * _NC + lax.axis_index("c")
        base = wid * tw
        pltpu.sync_copy(ia_hbm.at[pl.ds(wid * ncc, ncc)], ia_v)
        pltpu.sync_copy(ib_hbm.at[pl.ds(wid * ncc, ncc)], ib_v)
        for c in range(ncc):
            da = pltpu.async_copy(ys_hbm.at[ia_v.at[c]], ra, ga)
            db = pltpu.async_copy(ys_hbm.at[ib_v.at[c]], rb, gb)
            da.wait()
            db.wait()

            def add_body(k, carry):
                r = k // cols
                j = (k - r * cols) * 16
                ra[r, pl.ds(j, 16)] += rb[r, pl.ds(j, 16)]
                return carry

            lax.fori_loop(0, nvec, add_body, 0)
            do = pltpu.async_copy(
                ra, out_hbm.at[pl.ds(base + c * _CCH, _CCH)], osem)
            do.wait()

    return pl.kernel(
        body,
        out_type=jax.ShapeDtypeStruct((t, d), jnp.float32),
        mesh=mesh,
        scratch_types=[
            pltpu.VMEM((ncc, _CCH), jnp.int32),
            pltpu.VMEM((ncc, _CCH), jnp.int32),
            pltpu.VMEM((_CCH, d), jnp.float32),
            pltpu.VMEM((_CCH, d), jnp.float32),
            pltpu.SemaphoreType.DMA,
            pltpu.SemaphoreType.DMA,
            pltpu.SemaphoreType.DMA,
        ],
    )(ys, ia2, ib2)


def kernel(x, gate_w, gate_weights, up_weights, down_weights):
    b, s, embed = x.shape
    e = gate_w.shape[1]
    t = b * s
    xf = x.reshape(t, embed)

    logits = xf @ gate_w
    probs = jax.nn.softmax(logits, axis=-1)
    top_k_probs, top_k_indices = jax.lax.top_k(probs, 2)
    topk = top_k_indices.shape[-1]
    p = t * topk

    nb = p // BM
    nw = nb + e - 1

    flat_e = top_k_indices.reshape(-1).astype(jnp.int32)
    sort_idx = jnp.argsort(flat_e).astype(jnp.int32)
    st = sort_idx // topk
    sp = top_k_probs.reshape(-1)[sort_idx].reshape(p, 1)
    counts = jnp.bincount(flat_e, length=e).astype(jnp.int32)
    off = jnp.concatenate([jnp.zeros((1,), jnp.int32),
                           jnp.cumsum(counts).astype(jnp.int32)])
    fb = off[:-1] // BM
    lb = jnp.maximum(off[1:] - 1, 0) // BM
    nblk = jnp.where(counts > 0, lb - fb + 1, 0).astype(jnp.int32)
    istart = jnp.concatenate([jnp.zeros((1,), jnp.int32),
                              jnp.cumsum(nblk).astype(jnp.int32)])
    total = istart[e]
    wi_exp = jnp.repeat(jnp.arange(e, dtype=jnp.int32), nblk,
                        total_repeat_length=nw)
    valid = jnp.arange(nw, dtype=jnp.int32) < total
    wi_exp = jnp.where(valid, wi_exp, e - 1)
    wi_blk = fb[wi_exp] + jnp.arange(nw, dtype=jnp.int32) - istart[wi_exp]
    wi_blk = jnp.where(valid, wi_blk, nb - 1)
    row0 = wi_blk * BM
    lo_g = jnp.maximum(off[wi_exp], row0)
    hi_g = jnp.minimum(off[wi_exp + 1], row0 + BM)
    wi_lo = jnp.where(valid, lo_g - row0, 0).astype(jnp.int32)
    wi_hi = jnp.where(valid, hi_g - row0, 0).astype(jnp.int32)
    wi_first = jnp.where(valid & (wi_lo == 0), 1, 0).astype(jnp.int32)

    xs = _sc_gather(xf, st)

    ys = _grouped_ffn(xs, sp, gate_weights, up_weights, down_weights,
                      wi_blk, wi_exp, wi_lo, wi_hi, wi_first, nw)

    inv = jnp.zeros((p,), jnp.int32).at[sort_idx].set(
        jnp.arange(p, dtype=jnp.int32))
    ia = inv[0::topk]
    ib = inv[1::topk]
    out = _sc_combine(ys, ia, ib)
    return out.reshape(b, s, embed)
